# baseline (speedup 1.0000x reference)
import jax
import jax.numpy as jnp
from jax.experimental import pallas as pl
from jax.experimental.pallas import tpu as pltpu

_B, _L, _D = 16, 2048, 512


def _mlp_softmax_block(x_ref, w1_ref, b1_ref, w2_ref, b2_ref, w3_ref, b3_ref,
                       w4_ref, b4_ref, out_ref):
    x = x_ref[0]
    h = jnp.tanh(jnp.dot(x.astype(jnp.bfloat16),
                         w1_ref[...].astype(jnp.bfloat16),
                         preferred_element_type=jnp.float32)
                 + b1_ref[...])
    h = jnp.tanh(jnp.dot(h, w2_ref[...], preferred_element_type=jnp.float32)
                 + b2_ref[...])
    h = jnp.tanh(jnp.dot(h, w3_ref[...], preferred_element_type=jnp.float32)
                 + b3_ref[...])
    r = jax.lax.dot_general(w4_ref[...], h, (((0,), (1,)), ((), ())),
                            preferred_element_type=jnp.float32) + b4_ref[...]
    m = jnp.max(r)
    e = jnp.exp(r - m)
    out_ref[0] = e / jnp.sum(e)


def kernel(x, W1, b1, W2, b2, W3, b3, W4, b4):
    B, L, d = x.shape
    N = B * L
    full = lambda shape: pl.BlockSpec(shape, lambda i: (0,) * len(shape))
    res = pl.pallas_call(
        _mlp_softmax_block,
        grid=(B,),
        in_specs=[
            pl.BlockSpec((1, L, d), lambda i: (i, 0, 0)),
            full((d, 128)), full((1, 128)),
            full((128, 64)), full((1, 64)),
            full((64, 16)), full((1, 16)),
            full((16, 1)), full((1, 1)),
        ],
        out_specs=pl.BlockSpec((1, 1, L), lambda i: (i, 0, 0)),
        out_shape=jax.ShapeDtypeStruct((B, 1, L), jnp.float32),
    )(x, W1, b1.reshape(1, -1), W2, b2.reshape(1, -1),
      W3, b3.reshape(1, -1), W4, b4.reshape(1, -1))
    res_flat = res.reshape(N)
    sizes = jnp.full((B,), L, dtype=jnp.int32)
    return (res_flat, sizes)


# speedup vs baseline: 1.1058x; 1.1058x over previous
import jax
import jax.numpy as jnp
from jax.experimental import pallas as pl
from jax.experimental.pallas import tpu as pltpu

_B, _L, _D = 16, 2048, 512


def _mlp_softmax_block(xa_ref, xb_ref, w1a_ref, w1b_ref, b1_ref,
                       w2_ref, b2_ref, w3_ref, b3_ref,
                       w4_ref, b4_ref, out_ref):
    h = jnp.tanh(
        jnp.dot(xa_ref[0], w1a_ref[...], preferred_element_type=jnp.float32)
        + jnp.dot(xb_ref[0], w1b_ref[...], preferred_element_type=jnp.float32)
        + b1_ref[...])
    h = jnp.tanh(jnp.dot(h, w2_ref[...], preferred_element_type=jnp.float32)
                 + b2_ref[...])
    h = jnp.tanh(jnp.dot(h, w3_ref[...], preferred_element_type=jnp.float32)
                 + b3_ref[...])
    r = jax.lax.dot_general(w4_ref[...], h, (((0,), (1,)), ((), ())),
                            preferred_element_type=jnp.float32) + b4_ref[...]
    m = jnp.max(r)
    e = jnp.exp(r - m)
    out_ref[0] = e / jnp.sum(e)


def kernel(x, W1, b1, W2, b2, W3, b3, W4, b4):
    B, L, d = x.shape
    N = B * L
    hd = d // 2
    full = lambda shape: pl.BlockSpec(shape, lambda i: (0,) * len(shape))
    res = pl.pallas_call(
        _mlp_softmax_block,
        grid=(B,),
        in_specs=[
            pl.BlockSpec((1, L, hd), lambda i: (i, 0, 0)),
            pl.BlockSpec((1, L, hd), lambda i: (i, 0, 1)),
            pl.BlockSpec((hd, 128), lambda i: (0, 0)),
            pl.BlockSpec((hd, 128), lambda i: (1, 0)),
            full((1, 128)),
            full((128, 64)), full((1, 64)),
            full((64, 16)), full((1, 16)),
            full((16, 1)), full((1, 1)),
        ],
        out_specs=pl.BlockSpec((1, 1, L), lambda i: (i, 0, 0)),
        out_shape=jax.ShapeDtypeStruct((B, 1, L), jnp.float32),
    )(x, x, W1, W1, b1.reshape(1, -1), W2, b2.reshape(1, -1),
      W3, b3.reshape(1, -1), W4, b4.reshape(1, -1))
    res_flat = res.reshape(N)
    sizes = jnp.full((B,), L, dtype=jnp.int32)
    return (res_flat, sizes)
